# baseline (speedup 1.0000x reference)
import jax
import jax.numpy as jnp
from jax import lax
from jax.experimental import pallas as pl
from jax.experimental.pallas import tpu as pltpu
from jax.experimental.pallas import tpu_sc as plsc

DIM_ = 2048
NE_ = 64
NK_ = 8
NG_ = 8
GSZ_ = NE_ // NG_
NTG_ = 4
SCALE_ = 2.5
NT_ = 8192

_TILE = 512
_NSC = 2
_NSS = 16
_NW = _NSC * _NSS
_TW = NT_ // _NW
_NLG = _TW // 16
_IMIN = -(2**31)


def _score_body(x_ref, w_ref, b_ref, s_ref):
    logits = lax.dot_general(
        w_ref[...], x_ref[...], (((1,), (1,)), ((), ())),
        preferred_element_type=jnp.float32,
    )
    s_ref[...] = jax.nn.sigmoid(logits) + b_ref[...]


def _scores(x, weight, b2):
    n_tiles = NT_ // _TILE
    return pl.pallas_call(
        _score_body,
        grid=(n_tiles,),
        in_specs=[
            pl.BlockSpec((_TILE, DIM_), lambda i: (i, 0)),
            pl.BlockSpec((NE_, DIM_), lambda i: (0, 0)),
            pl.BlockSpec((NE_, 1), lambda i: (0, 0)),
        ],
        out_specs=pl.BlockSpec((NE_, _TILE), lambda i: (0, i)),
        out_shape=jax.ShapeDtypeStruct((NE_, NT_), jnp.float32),
        compiler_params=pltpu.CompilerParams(
            dimension_semantics=("arbitrary",),
        ),
    )(x, weight, b2)


def _sc_route_body(s_hbm, b_hbm, wout_hbm, iout_hbm,
                   s_v, b_v, t_v, a_v, wout_v, iout_v):
    wid = lax.axis_index("s") * _NSC + lax.axis_index("c")
    base = wid * _TW
    pltpu.sync_copy(s_hbm.at[:, pl.ds(base, _TW)], s_v)
    pltpu.sync_copy(b_hbm, b_v)
    iota16 = lax.iota(jnp.int32, 16)

    def do_group(g, carry):
        o = g * 16
        col = iota16 + o
        ninf = jnp.full((16,), -jnp.inf, jnp.float32)
        gs, tmax, targ = [], [], []
        for grp in range(NG_):
            e0 = grp * GSZ_
            m1 = s_v[e0, pl.ds(o, 16)]
            idx1 = jnp.full((16,), e0, jnp.int32)
            m2 = ninf
            for j in range(1, GSZ_):
                v = s_v[e0 + j, pl.ds(o, 16)]
                cond = v > m1
                m2 = jnp.maximum(m2, jnp.minimum(m1, v))
                m1 = jnp.maximum(m1, v)
                idx1 = jnp.where(cond, e0 + j, idx1)
            gs.append(m1 + m2)
            tmax.append(m1)
            targ.append(idx1)
        for gi in range(NG_):
            rank = jnp.zeros((16,), jnp.int32)
            for h in range(NG_):
                if h == gi:
                    continue
                beats = gs[h] >= gs[gi] if h < gi else gs[h] > gs[gi]
                rank = rank + jnp.where(beats, 1, 0)
            t_v[gi, :] = jnp.where(rank < NTG_, tmax[gi], -jnp.inf)
            a_v[gi, :] = targ[gi]
        wk_list, idx_list = [], []
        for _ in range(NK_):
            pairs = []
            for grp in range(0, NG_, 2):
                ta, tb = t_v[grp, :], t_v[grp + 1, :]
                cond = tb > ta
                pairs.append((jnp.maximum(ta, tb),
                              jnp.where(cond, grp + 1, grp)))
            while len(pairs) > 1:
                nxt = []
                for a in range(0, len(pairs), 2):
                    (ma, wa), (mb, wb) = pairs[a], pairs[a + 1]
                    cond = mb > ma
                    nxt.append((jnp.maximum(ma, mb), jnp.where(cond, wb, wa)))
                pairs = nxt
            m, w = pairs[0]
            idx = plsc.load_gather(a_v, [w, iota16])
            bval = plsc.load_gather(b_v, [idx])
            wk_list.append(m - bval)
            idx_list.append(idx)
            plsc.store_scatter(s_v, [idx, col], ninf)
            eb = w * GSZ_
            nm = plsc.load_gather(s_v, [eb, col])
            nidx = eb
            for j in range(1, GSZ_):
                ej = eb + j
                vj = plsc.load_gather(s_v, [ej, col])
                cond = vj > nm
                nm = jnp.maximum(nm, vj)
                nidx = jnp.where(cond, ej, nidx)
            plsc.store_scatter(t_v, [w, iota16], nm)
            plsc.store_scatter(a_v, [w, iota16], nidx)
        wsum = wk_list[0]
        for k in range(1, NK_):
            wsum = wsum + wk_list[k]
        scale = jnp.float32(SCALE_) / wsum
        rows = iota16 + o
        for k in range(NK_):
            colk = jnp.full((16,), k, jnp.int32)
            plsc.store_scatter(wout_v, [rows, colk], wk_list[k] * scale)
            plsc.store_scatter(iout_v, [rows, colk], idx_list[k])
        return carry

    lax.fori_loop(0, _NLG, do_group, 0)
    pltpu.sync_copy(wout_v, wout_hbm.at[pl.ds(base, _TW), :])
    pltpu.sync_copy(iout_v, iout_hbm.at[pl.ds(base, _TW), :])


def kernel(x, weight, e_score_correction_bias):
    b2 = e_score_correction_bias.reshape(NE_, 1)
    s = _scores(x, weight, b2)
    route = pl.kernel(
        _sc_route_body,
        out_type=[
            jax.ShapeDtypeStruct((NT_, NK_), jnp.float32),
            jax.ShapeDtypeStruct((NT_, NK_), jnp.int32),
        ],
        mesh=plsc.VectorSubcoreMesh(core_axis_name="c", subcore_axis_name="s"),
        compiler_params=pltpu.CompilerParams(needs_layout_passes=False),
        scratch_types=[
            pltpu.VMEM((NE_, _TW), jnp.float32),
            pltpu.VMEM((NE_,), jnp.float32),
            pltpu.VMEM((NG_, 16), jnp.float32),
            pltpu.VMEM((NG_, 16), jnp.int32),
            pltpu.VMEM((_TW, NK_), jnp.float32),
            pltpu.VMEM((_TW, NK_), jnp.int32),
        ],
    )
    weights, indices = route(s, e_score_correction_bias)
    return weights, indices


# speedup vs baseline: 1.0139x; 1.0139x over previous
import jax
import jax.numpy as jnp
from jax import lax
from jax.experimental import pallas as pl
from jax.experimental.pallas import tpu as pltpu
from jax.experimental.pallas import tpu_sc as plsc

DIM_ = 2048
NE_ = 64
NK_ = 8
NG_ = 8
GSZ_ = NE_ // NG_
NTG_ = 4
SCALE_ = 2.5
NT_ = 8192

_TILE = 512
_NSC = 2
_NSS = 16
_NW = _NSC * _NSS
_TW = NT_ // _NW
_NLG = _TW // 16
_IMIN = -(2**31)


def _score_body(x_ref, w_ref, b_ref, s_ref):
    logits = lax.dot_general(
        w_ref[...], x_ref[...], (((1,), (1,)), ((), ())),
        preferred_element_type=jnp.float32,
    )
    s_ref[...] = jax.nn.sigmoid(logits) + b_ref[...]


def _scores(x, weight, b2, chunk, nchunks):
    ct = NT_ // nchunks
    n_tiles = ct // _TILE
    t0 = chunk * n_tiles
    return pl.pallas_call(
        _score_body,
        grid=(n_tiles,),
        in_specs=[
            pl.BlockSpec((_TILE, DIM_), lambda i: (t0 + i, 0)),
            pl.BlockSpec((NE_, DIM_), lambda i: (0, 0)),
            pl.BlockSpec((NE_, 1), lambda i: (0, 0)),
        ],
        out_specs=pl.BlockSpec((NE_, _TILE), lambda i: (0, i)),
        out_shape=jax.ShapeDtypeStruct((NE_, ct), jnp.float32),
        compiler_params=pltpu.CompilerParams(
            dimension_semantics=("arbitrary",),
        ),
    )(x, weight, b2)


def _sc_route_body(s_hbm, b_hbm, wout_hbm, iout_hbm,
                   s_v, b_v, t_v, a_v, wout_v, iout_v):
    tw = s_hbm.shape[1] // _NW
    nlg = tw // 16
    wid = lax.axis_index("s") * _NSC + lax.axis_index("c")
    base = wid * tw
    pltpu.sync_copy(s_hbm.at[:, pl.ds(base, tw)], s_v)
    pltpu.sync_copy(b_hbm, b_v)
    iota16 = lax.iota(jnp.int32, 16)

    def do_group(g, carry):
        o = g * 16
        col = iota16 + o
        ninf = jnp.full((16,), -jnp.inf, jnp.float32)
        gs, tmax, targ = [], [], []
        for grp in range(NG_):
            e0 = grp * GSZ_
            m1 = s_v[e0, pl.ds(o, 16)]
            idx1 = jnp.full((16,), e0, jnp.int32)
            m2 = ninf
            for j in range(1, GSZ_):
                v = s_v[e0 + j, pl.ds(o, 16)]
                cond = v > m1
                m2 = jnp.maximum(m2, jnp.minimum(m1, v))
                m1 = jnp.maximum(m1, v)
                idx1 = jnp.where(cond, e0 + j, idx1)
            gs.append(m1 + m2)
            tmax.append(m1)
            targ.append(idx1)
        for gi in range(NG_):
            rank = jnp.zeros((16,), jnp.int32)
            for h in range(NG_):
                if h == gi:
                    continue
                beats = gs[h] >= gs[gi] if h < gi else gs[h] > gs[gi]
                rank = rank + jnp.where(beats, 1, 0)
            t_v[gi, :] = jnp.where(rank < NTG_, tmax[gi], -jnp.inf)
            a_v[gi, :] = targ[gi]
        wk_list, idx_list = [], []
        for _ in range(NK_):
            pairs = []
            for grp in range(0, NG_, 2):
                ta, tb = t_v[grp, :], t_v[grp + 1, :]
                cond = tb > ta
                pairs.append((jnp.maximum(ta, tb),
                              jnp.where(cond, grp + 1, grp)))
            while len(pairs) > 1:
                nxt = []
                for a in range(0, len(pairs), 2):
                    (ma, wa), (mb, wb) = pairs[a], pairs[a + 1]
                    cond = mb > ma
                    nxt.append((jnp.maximum(ma, mb), jnp.where(cond, wb, wa)))
                pairs = nxt
            m, w = pairs[0]
            idx = plsc.load_gather(a_v, [w, iota16])
            bval = plsc.load_gather(b_v, [idx])
            wk_list.append(m - bval)
            idx_list.append(idx)
            plsc.store_scatter(s_v, [idx, col], ninf)
            eb = w * GSZ_
            nm = plsc.load_gather(s_v, [eb, col])
            nidx = eb
            for j in range(1, GSZ_):
                ej = eb + j
                vj = plsc.load_gather(s_v, [ej, col])
                cond = vj > nm
                nm = jnp.maximum(nm, vj)
                nidx = jnp.where(cond, ej, nidx)
            plsc.store_scatter(t_v, [w, iota16], nm)
            plsc.store_scatter(a_v, [w, iota16], nidx)
        wsum = wk_list[0]
        for k in range(1, NK_):
            wsum = wsum + wk_list[k]
        scale = jnp.float32(SCALE_) / wsum
        rows = iota16 + o
        for k in range(NK_):
            colk = jnp.full((16,), k, jnp.int32)
            plsc.store_scatter(wout_v, [rows, colk], wk_list[k] * scale)
            plsc.store_scatter(iout_v, [rows, colk], idx_list[k])
        return carry

    lax.fori_loop(0, nlg, do_group, 0)
    pltpu.sync_copy(wout_v, wout_hbm.at[pl.ds(base, tw), :])
    pltpu.sync_copy(iout_v, iout_hbm.at[pl.ds(base, tw), :])


_NCHUNKS = 2


def kernel(x, weight, e_score_correction_bias):
    b2 = e_score_correction_bias.reshape(NE_, 1)
    ct = NT_ // _NCHUNKS
    tw = ct // _NW
    route = pl.kernel(
        _sc_route_body,
        out_type=[
            jax.ShapeDtypeStruct((ct, NK_), jnp.float32),
            jax.ShapeDtypeStruct((ct, NK_), jnp.int32),
        ],
        mesh=plsc.VectorSubcoreMesh(core_axis_name="c", subcore_axis_name="s"),
        compiler_params=pltpu.CompilerParams(needs_layout_passes=False),
        scratch_types=[
            pltpu.VMEM((NE_, tw), jnp.float32),
            pltpu.VMEM((NE_,), jnp.float32),
            pltpu.VMEM((NG_, 16), jnp.float32),
            pltpu.VMEM((NG_, 16), jnp.int32),
            pltpu.VMEM((tw, NK_), jnp.float32),
            pltpu.VMEM((tw, NK_), jnp.int32),
        ],
    )
    parts = []
    for c in range(_NCHUNKS):
        s = _scores(x, weight, b2, c, _NCHUNKS)
        parts.append(route(s, e_score_correction_bias))
    weights = jnp.concatenate([p[0] for p in parts], axis=0)
    indices = jnp.concatenate([p[1] for p in parts], axis=0)
    return weights, indices


# speedup vs baseline: 1.1084x; 1.0932x over previous
import jax
import jax.numpy as jnp
from jax import lax
from jax.experimental import pallas as pl
from jax.experimental.pallas import tpu as pltpu
from jax.experimental.pallas import tpu_sc as plsc

DIM_ = 2048
NE_ = 64
NK_ = 8
NG_ = 8
GSZ_ = NE_ // NG_
NTG_ = 4
SCALE_ = 2.5
NT_ = 8192

_TILE = 512
_NSC = 2
_NSS = 16
_NW = _NSC * _NSS
_TW = NT_ // _NW
_NLG = _TW // 16
_IMIN = -(2**31)


def _score_body(x_ref, w_ref, b_ref, s_ref):
    logits = lax.dot_general(
        w_ref[...], x_ref[...], (((1,), (1,)), ((), ())),
        preferred_element_type=jnp.float32,
    )
    s_ref[...] = jax.nn.sigmoid(logits) + b_ref[...]


def _scores(x, weight, b2, chunk, nchunks):
    ct = NT_ // nchunks
    n_tiles = ct // _TILE
    t0 = chunk * n_tiles
    return pl.pallas_call(
        _score_body,
        grid=(n_tiles,),
        in_specs=[
            pl.BlockSpec((_TILE, DIM_), lambda i: (t0 + i, 0)),
            pl.BlockSpec((NE_, DIM_), lambda i: (0, 0)),
            pl.BlockSpec((NE_, 1), lambda i: (0, 0)),
        ],
        out_specs=pl.BlockSpec((NE_, _TILE), lambda i: (0, i)),
        out_shape=jax.ShapeDtypeStruct((NE_, ct), jnp.float32),
        compiler_params=pltpu.CompilerParams(
            dimension_semantics=("arbitrary",),
        ),
    )(x, weight, b2)


def _sc_route_body(s_hbm, b_hbm, wout_hbm, iout_hbm,
                   s_v, b_v, t_v, a_v, wout_v, iout_v):
    tw = s_hbm.shape[1] // _NW
    nlg = tw // 16
    wid = lax.axis_index("s") * _NSC + lax.axis_index("c")
    base = wid * tw
    pltpu.sync_copy(s_hbm.at[:, pl.ds(base, tw)], s_v)
    pltpu.sync_copy(b_hbm, b_v)
    iota16 = lax.iota(jnp.int32, 16)

    def do_group(g, carry):
        o = g * 16
        col = iota16 + o
        ninf = jnp.full((16,), -jnp.inf, jnp.float32)
        gs, tmax, targ = [], [], []
        for grp in range(NG_):
            e0 = grp * GSZ_
            m1 = s_v[e0, pl.ds(o, 16)]
            idx1 = jnp.full((16,), e0, jnp.int32)
            m2 = ninf
            for j in range(1, GSZ_):
                v = s_v[e0 + j, pl.ds(o, 16)]
                cond = v > m1
                m2 = jnp.maximum(m2, jnp.minimum(m1, v))
                m1 = jnp.maximum(m1, v)
                idx1 = jnp.where(cond, e0 + j, idx1)
            gs.append(m1 + m2)
            tmax.append(m1)
            targ.append(idx1)
        for gi in range(NG_):
            rank = jnp.zeros((16,), jnp.int32)
            for h in range(NG_):
                if h == gi:
                    continue
                beats = gs[h] >= gs[gi] if h < gi else gs[h] > gs[gi]
                rank = rank + jnp.where(beats, 1, 0)
            t_v[gi, :] = jnp.where(rank < NTG_, tmax[gi], -jnp.inf)
            a_v[gi, :] = targ[gi]
        wk_list, idx_list = [], []
        for _ in range(NK_):
            pairs = []
            for grp in range(0, NG_, 2):
                ta, tb = t_v[grp, :], t_v[grp + 1, :]
                cond = tb > ta
                pairs.append((jnp.maximum(ta, tb),
                              jnp.where(cond, grp + 1, grp)))
            while len(pairs) > 1:
                nxt = []
                for a in range(0, len(pairs), 2):
                    (ma, wa), (mb, wb) = pairs[a], pairs[a + 1]
                    cond = mb > ma
                    nxt.append((jnp.maximum(ma, mb), jnp.where(cond, wb, wa)))
                pairs = nxt
            m, w = pairs[0]
            idx = plsc.load_gather(a_v, [w, iota16])
            bval = plsc.load_gather(b_v, [idx])
            wk_list.append(m - bval)
            idx_list.append(idx)
            plsc.store_scatter(s_v, [idx, col], ninf)
            eb = w * GSZ_
            nm = plsc.load_gather(s_v, [eb, col])
            nidx = eb
            for j in range(1, GSZ_):
                ej = eb + j
                vj = plsc.load_gather(s_v, [ej, col])
                cond = vj > nm
                nm = jnp.maximum(nm, vj)
                nidx = jnp.where(cond, ej, nidx)
            plsc.store_scatter(t_v, [w, iota16], nm)
            plsc.store_scatter(a_v, [w, iota16], nidx)
        wsum = wk_list[0]
        for k in range(1, NK_):
            wsum = wsum + wk_list[k]
        scale = jnp.float32(SCALE_) / wsum
        for k in range(NK_):
            wout_v[k, pl.ds(o, 16)] = wk_list[k] * scale
            iout_v[k, pl.ds(o, 16)] = idx_list[k]
        return carry

    lax.fori_loop(0, nlg, do_group, 0)
    pltpu.sync_copy(wout_v, wout_hbm.at[:, pl.ds(base, tw)])
    pltpu.sync_copy(iout_v, iout_hbm.at[:, pl.ds(base, tw)])


_NCHUNKS = 2


def kernel(x, weight, e_score_correction_bias):
    b2 = e_score_correction_bias.reshape(NE_, 1)
    ct = NT_ // _NCHUNKS
    tw = ct // _NW
    route = pl.kernel(
        _sc_route_body,
        out_type=[
            jax.ShapeDtypeStruct((NK_, ct), jnp.float32),
            jax.ShapeDtypeStruct((NK_, ct), jnp.int32),
        ],
        mesh=plsc.VectorSubcoreMesh(core_axis_name="c", subcore_axis_name="s"),
        compiler_params=pltpu.CompilerParams(needs_layout_passes=False),
        scratch_types=[
            pltpu.VMEM((NE_, tw), jnp.float32),
            pltpu.VMEM((NE_,), jnp.float32),
            pltpu.VMEM((NG_, 16), jnp.float32),
            pltpu.VMEM((NG_, 16), jnp.int32),
            pltpu.VMEM((NK_, tw), jnp.float32),
            pltpu.VMEM((NK_, tw), jnp.int32),
        ],
    )
    parts = []
    for c in range(_NCHUNKS):
        s = _scores(x, weight, b2, c, _NCHUNKS)
        parts.append(route(s, e_score_correction_bias))
    weights = jnp.concatenate([p[0] for p in parts], axis=1)
    indices = jnp.concatenate([p[1] for p in parts], axis=1)
    return weights.T, indices.T
